# Initial kernel scaffold; baseline (speedup 1.0000x reference)
import jax
import jax.numpy as jnp
from jax.experimental import pallas as pl


def kernel(card_indices, table):
    raise NotImplementedError("write your pallas kernel here")


# speedup vs baseline: 2.9854x; 2.9854x over previous
import functools

import jax
import jax.numpy as jnp
from jax import lax
from jax.experimental import pallas as pl
from jax.experimental.pallas import tpu as pltpu
from jax.experimental.pallas import tpu_sc as plsc

NUM_CARDS = 100000
EMBED_DIM = 32
BATCH = 16384
HIST = 50

ROWS_TOTAL = BATCH * HIST
IDX_MINOR = 128
IDX_ROWS = ROWS_TOTAL // IDX_MINOR

NUM_WORKERS = 32
IDX_ROWS_PER_W = IDX_ROWS // NUM_WORKERS
ROWS_PER_W = ROWS_TOTAL // NUM_WORKERS

CHUNK_IDX_ROWS = 10
CHUNK_ROWS = CHUNK_IDX_ROWS * IDX_MINOR
NUM_CHUNKS = IDX_ROWS_PER_W // CHUNK_IDX_ROWS

_mesh = plsc.VectorSubcoreMesh(core_axis_name="c", subcore_axis_name="s")


@functools.partial(
    pl.kernel,
    mesh=_mesh,
    compiler_params=pltpu.CompilerParams(use_tc_tiling_on_sc=False),
    out_type=jax.ShapeDtypeStruct((ROWS_TOTAL, EMBED_DIM), jnp.float32),
    scratch_types=[
        pltpu.VMEM((IDX_ROWS_PER_W, IDX_MINOR), jnp.int32),
        pltpu.VMEM((CHUNK_ROWS, EMBED_DIM), jnp.float32),
        pltpu.SemaphoreType.DMA,
    ],
)
def _gather_kernel(idx_hbm, table_hbm, out_hbm, idx_v, rows_v, gsem):
    nc = 2
    wid = lax.axis_index("s") * nc + lax.axis_index("c")

    pltpu.sync_copy(idx_hbm.at[pl.ds(wid * IDX_ROWS_PER_W, IDX_ROWS_PER_W)],
                    idx_v)

    out_base = wid * ROWS_PER_W

    def chunk_body(n, carry):
        ir = n * CHUNK_IDX_ROWS
        copies = [
            pltpu.async_copy(
                table_hbm.at[idx_v.at[ir + j]],
                rows_v.at[pl.ds(j * IDX_MINOR, IDX_MINOR)],
                gsem,
            )
            for j in range(CHUNK_IDX_ROWS)
        ]
        for c in copies:
            c.wait()
        pltpu.sync_copy(rows_v,
                        out_hbm.at[pl.ds(out_base + n * CHUNK_ROWS,
                                         CHUNK_ROWS)])
        return carry

    lax.fori_loop(0, NUM_CHUNKS, chunk_body, 0)


def kernel(card_indices, table):
    idx = card_indices.astype(jnp.int32).reshape(IDX_ROWS, IDX_MINOR)
    out = _gather_kernel(idx, table)
    return out.reshape(BATCH, HIST, EMBED_DIM)
